# Initial kernel scaffold; baseline (speedup 1.0000x reference)
import functools

import jax
import jax.numpy as jnp
from jax import lax
from jax.experimental import pallas as pl
from jax.experimental.pallas import tpu as pltpu
from jax.experimental.pallas import tpu_sc as plsc

_NC = 2
_NS = 16
_NW = _NC * _NS
_CH = 128
_NBUF = 4


@functools.lru_cache(maxsize=None)
def _build(B, V, D):
    assert B % (_NW * _CH) == 0
    b_per_w = B // _NW
    n_ch = b_per_w // _CH
    assert n_ch > _NBUF and n_ch % _NBUF == 0

    mesh = plsc.VectorSubcoreMesh(core_axis_name="c", subcore_axis_name="s")

    @functools.partial(
        pl.kernel,
        out_type=jax.ShapeDtypeStruct((B, D), jnp.float32),
        mesh=mesh,
        scratch_types=[
            pltpu.VMEM((n_ch, _CH), jnp.int32),
            pltpu.VMEM((_NBUF, _CH, D), jnp.float32),
            [pltpu.SemaphoreType.DMA] * _NBUF,
        ],
    )
    def gather_kernel(idx_hbm, table_hbm, out_hbm, idx_v, rows_v, sems):
        wid = lax.axis_index("s") * _NC + lax.axis_index("c")
        base = wid * b_per_w
        pltpu.sync_copy(idx_hbm.at[wid], idx_v)

        for b in range(_NBUF):
            pltpu.async_copy(table_hbm.at[idx_v.at[b]], rows_v.at[b], sems[b])

        @pl.loop(0, n_ch - _NBUF, step=_NBUF)
        def _steady(j):
            for b in range(_NBUF):
                c = j + b
                pltpu.make_async_copy(
                    table_hbm.at[idx_v.at[c]], rows_v.at[b], sems[b]
                ).wait()
                pltpu.sync_copy(
                    rows_v.at[b], out_hbm.at[pl.ds(base + c * _CH, _CH)]
                )
                pltpu.async_copy(
                    table_hbm.at[idx_v.at[c + _NBUF]], rows_v.at[b], sems[b]
                )

        for b in range(_NBUF):
            c = n_ch - _NBUF + b
            pltpu.make_async_copy(
                table_hbm.at[idx_v.at[c]], rows_v.at[b], sems[b]
            ).wait()
            pltpu.sync_copy(rows_v.at[b], out_hbm.at[pl.ds(base + c * _CH, _CH)])

    return gather_kernel


def kernel(indices_, table):
    Bb, H = indices_.shape
    V, D = table.shape
    B = Bb * H
    b_per_w = B // _NW
    idx3 = indices_.reshape(_NW, b_per_w // _CH, _CH).astype(jnp.int32)
    out = _build(B, V, D)(idx3, table)
    return out.reshape(Bb, H, D)


# speedup vs baseline: 1.8754x; 1.8754x over previous
import functools

import jax
import jax.numpy as jnp
from jax import lax
from jax.experimental import pallas as pl
from jax.experimental.pallas import tpu as pltpu
from jax.experimental.pallas import tpu_sc as plsc

_NC = 2
_NS = 16
_NW = _NC * _NS
_CH = 128
_NBUF = 4


@functools.lru_cache(maxsize=None)
def _build(B, V, D):
    assert B % (_NW * _CH) == 0
    b_per_w = B // _NW
    n_ch = b_per_w // _CH
    assert n_ch > _NBUF and n_ch % _NBUF == 0

    mesh = plsc.VectorSubcoreMesh(core_axis_name="c", subcore_axis_name="s")

    @functools.partial(
        pl.kernel,
        out_type=jax.ShapeDtypeStruct((B, D), jnp.float32),
        mesh=mesh,
        scratch_types=[
            pltpu.VMEM((n_ch, _CH), jnp.int32),
            pltpu.VMEM((_NBUF, _CH, D), jnp.float32),
            [pltpu.SemaphoreType.DMA] * _NBUF,
        ],
        compiler_params=pltpu.CompilerParams(use_tc_tiling_on_sc=False),
    )
    def gather_kernel(idx_hbm, table_hbm, out_hbm, idx_v, rows_v, sems):
        wid = lax.axis_index("s") * _NC + lax.axis_index("c")
        base = wid * b_per_w
        pltpu.sync_copy(idx_hbm.at[wid], idx_v)

        for b in range(_NBUF):
            pltpu.async_copy(table_hbm.at[idx_v.at[b]], rows_v.at[b], sems[b])

        @pl.loop(0, n_ch - _NBUF, step=_NBUF)
        def _steady(j):
            for b in range(_NBUF):
                c = j + b
                pltpu.make_async_copy(
                    table_hbm.at[idx_v.at[c]], rows_v.at[b], sems[b]
                ).wait()
                pltpu.sync_copy(
                    rows_v.at[b], out_hbm.at[pl.ds(base + c * _CH, _CH)]
                )
                pltpu.async_copy(
                    table_hbm.at[idx_v.at[c + _NBUF]], rows_v.at[b], sems[b]
                )

        for b in range(_NBUF):
            c = n_ch - _NBUF + b
            pltpu.make_async_copy(
                table_hbm.at[idx_v.at[c]], rows_v.at[b], sems[b]
            ).wait()
            pltpu.sync_copy(rows_v.at[b], out_hbm.at[pl.ds(base + c * _CH, _CH)])

    return gather_kernel


def kernel(indices_, table):
    Bb, H = indices_.shape
    V, D = table.shape
    B = Bb * H
    b_per_w = B // _NW
    idx3 = indices_.reshape(_NW, b_per_w // _CH, _CH).astype(jnp.int32)
    out = _build(B, V, D)(idx3, table)
    return out.reshape(Bb, H, D)
